# Initial kernel scaffold; baseline (speedup 1.0000x reference)
import jax
import jax.numpy as jnp
from jax.experimental import pallas as pl


def kernel(xi, memory, least_used_mem, Wq, bq, Wv, bv, Wg, bg, Wwg, bwg):
    raise NotImplementedError("write your pallas kernel here")


# speedup vs baseline: 10.9388x; 10.9388x over previous
import functools

import jax
import jax.numpy as jnp
from jax.experimental import pallas as pl
from jax.experimental.pallas import tpu as pltpu

_B, _I, _M, _W, _R, _K = 16, 512, 100000, 32, 4, 4
_C = _R * _K + 1
_DELTA = 1e-6
_BLK = 25000
_NMB = _M // _BLK
_NEG = -3.0e38


def _iface_body(xi_ref, wq_ref, bq_ref, wv_ref, bv_ref, wg_ref, bg_ref,
                wwg_ref, bwg_ref, qn_ref, v_ref, ww_ref):
    hi = jax.lax.Precision.HIGHEST
    xi = xi_ref[...]
    rq = jnp.dot(xi, wq_ref[...],
                 preferred_element_type=jnp.float32) + bq_ref[...]
    row = jax.lax.broadcasted_iota(jnp.int32, (_R * _W, _R * _W), 0) // _W
    col = jax.lax.broadcasted_iota(jnp.int32, (_R * _W, _R * _W), 1) // _W
    blockdiag = (row == col).astype(jnp.float32)
    ssq = jnp.dot(rq * rq, blockdiag, precision=hi,
                  preferred_element_type=jnp.float32)
    qn_ref[...] = rq / (jnp.sqrt(ssq) + _DELTA)
    v_ref[...] = jnp.dot(xi, wv_ref[...],
                         preferred_element_type=jnp.float32) + bv_ref[...]
    gates = jax.nn.sigmoid(
        jnp.dot(xi, wg_ref[...],
                preferred_element_type=jnp.float32) + bg_ref[...])
    wgate = jax.nn.sigmoid(
        jnp.dot(xi, wwg_ref[...],
                preferred_element_type=jnp.float32) + bwg_ref[...])
    ww_ref[...] = wgate * gates


def _scan_body(qn_ref, mem_ref, lu_ref, pos_ref, tv_ref, ti_ref):
    mb = pl.program_id(1)

    @pl.when(mb == 0)
    def _():
        tv_ref[...] = jnp.full((_R, _K), _NEG, jnp.float32)
        ti_ref[...] = jnp.zeros((_R, _K), jnp.int32)

    mem = mem_ref[0]
    qn = qn_ref[0]
    ssq = jnp.sum(mem * mem, axis=1, keepdims=True)
    mn = mem / (jnp.sqrt(ssq) + _DELTA)
    sims = jax.lax.dot_general(qn, mn, (((1,), (1,)), ((), ())),
                               preferred_element_type=jnp.float32)

    col = jax.lax.broadcasted_iota(jnp.int32, (_R, _BLK), 1) + mb * _BLK
    s = sims
    blk_v, blk_i = [], []
    for _ in range(_K):
        v = jnp.max(s, axis=1, keepdims=True)
        i = jnp.min(jnp.where(s == v, col, jnp.int32(2 ** 30)),
                    axis=1, keepdims=True)
        blk_v.append(v)
        blk_i.append(i)
        s = jnp.where(col == i, _NEG, s)

    cv = jnp.concatenate([tv_ref[...]] + blk_v, axis=1)
    ci = jnp.concatenate([ti_ref[...]] + blk_i, axis=1)
    col8 = jax.lax.broadcasted_iota(jnp.int32, (_R, 2 * _K), 1)
    nv, ni = [], []
    for _ in range(_K):
        v = jnp.max(cv, axis=1, keepdims=True)
        p = jnp.min(jnp.where(cv == v, col8, jnp.int32(2 * _K)),
                    axis=1, keepdims=True)
        sel = col8 == p
        i = jnp.sum(jnp.where(sel, ci, 0), axis=1, keepdims=True)
        nv.append(v)
        ni.append(i)
        cv = jnp.where(sel, _NEG, cv)
    tv_ref[...] = jnp.concatenate(nv, axis=1)
    ti_ref[...] = jnp.concatenate(ni, axis=1)

    @pl.when(mb == _NMB - 1)
    def _():
        for r in range(_R):
            pos_ref[0, 0, r * _K:(r + 1) * _K] = ti_ref[r, :]
        pos_ref[0, 0, _R * _K:_R * _K + 1] = lu_ref[0, 0, :]


def _gather_body(pos_sref, mem_ref, qn_ref, wv_ref, ww_ref, out_ref, vis_ref):
    del pos_sref
    b = pl.program_id(0)
    c = pl.program_id(1)
    w = ww_ref[b, c]
    row = mem_ref[0, 0]
    vis_ref[pl.ds(c, 1), :] = row * (1.0 - w) + w * wv_ref[0]

    @pl.when(c == _C - 1)
    def _():
        vis = vis_ref[...]
        ssq = jnp.sum(vis * vis, axis=1, keepdims=True)
        vn = vis / (jnp.sqrt(ssq) + _DELTA)
        q = qn_ref[0]
        rs = jax.lax.dot_general(q, vn, (((1,), (1,)), ((), ())),
                                 preferred_element_type=jnp.float32)
        m = jnp.max(rs, axis=1, keepdims=True)
        e = jnp.exp(rs - m)
        p = e / jnp.sum(e, axis=1, keepdims=True)
        out_ref[0] = jnp.dot(p, vis, preferred_element_type=jnp.float32)


@jax.jit
def kernel(xi, memory, least_used_mem, Wq, bq, Wv, bv, Wg, bg, Wwg, bwg):
    f32 = jnp.float32
    qn_flat, wv, ww = pl.pallas_call(
        _iface_body,
        out_shape=[
            jax.ShapeDtypeStruct((_B, _R * _W), f32),
            jax.ShapeDtypeStruct((_B, _W), f32),
            jax.ShapeDtypeStruct((_B, _C), f32),
        ],
    )(xi, Wq, bq.reshape(1, -1), Wv, bv.reshape(1, -1), Wg, bg.reshape(1, -1),
      Wwg, bwg.reshape(1, -1))

    qn3 = qn_flat.reshape(_B, _R, _W)
    lu3 = least_used_mem.reshape(_B, 1, 1)

    pos3 = pl.pallas_call(
        _scan_body,
        grid=(_B, _NMB),
        in_specs=[
            pl.BlockSpec((1, _R, _W), lambda b, mb: (b, 0, 0)),
            pl.BlockSpec((1, _BLK, _W), lambda b, mb: (b, mb, 0)),
            pl.BlockSpec((1, 1, 1), lambda b, mb: (b, 0, 0)),
        ],
        out_specs=pl.BlockSpec((1, 1, _C), lambda b, mb: (b, 0, 0)),
        out_shape=jax.ShapeDtypeStruct((_B, 1, _C), jnp.int32),
        scratch_shapes=[
            pltpu.VMEM((_R, _K), f32),
            pltpu.VMEM((_R, _K), jnp.int32),
        ],
    )(qn3, memory, lu3)
    positions = pos3.reshape(_B, _C)

    read_vectors = pl.pallas_call(
        _gather_body,
        grid_spec=pltpu.PrefetchScalarGridSpec(
            num_scalar_prefetch=1,
            grid=(_B, _C),
            in_specs=[
                pl.BlockSpec((1, 1, 1, _W), lambda b, c, pos: (b, pos[b, c], 0, 0)),
                pl.BlockSpec((1, _R, _W), lambda b, c, pos: (b, 0, 0)),
                pl.BlockSpec((1, 1, _W), lambda b, c, pos: (b, 0, 0)),
                pl.BlockSpec(memory_space=pltpu.SMEM),
            ],
            out_specs=pl.BlockSpec((1, _R, _W), lambda b, c, pos: (b, 0, 0)),
            scratch_shapes=[pltpu.VMEM((_C, _W), f32)],
        ),
        out_shape=jax.ShapeDtypeStruct((_B, _R, _W), f32),
    )(positions, memory.reshape(_B, _M, 1, _W), qn3, wv.reshape(_B, 1, _W), ww)

    return read_vectors


# speedup vs baseline: 11.9765x; 1.0949x over previous
import functools

import jax
import jax.numpy as jnp
from jax.experimental import pallas as pl
from jax.experimental.pallas import tpu as pltpu

_B, _I, _M, _W, _R, _K = 16, 512, 100000, 32, 4, 4
_C = _R * _K + 1
_DELTA = 1e-6
_BLK = 25000
_NMB = _M // _BLK
_NEG = -3.0e38


def _iface_body(xi_ref, wq_ref, bq_ref, wv_ref, bv_ref, wg_ref, bg_ref,
                wwg_ref, bwg_ref, qn_ref, v_ref, ww_ref):
    hi = jax.lax.Precision.HIGHEST
    xi = xi_ref[...]
    rq = jnp.dot(xi, wq_ref[...],
                 preferred_element_type=jnp.float32) + bq_ref[...]
    row = jax.lax.broadcasted_iota(jnp.int32, (_R * _W, _R * _W), 0) // _W
    col = jax.lax.broadcasted_iota(jnp.int32, (_R * _W, _R * _W), 1) // _W
    blockdiag = (row == col).astype(jnp.float32)
    ssq = jnp.dot(rq * rq, blockdiag, precision=hi,
                  preferred_element_type=jnp.float32)
    qn_ref[...] = rq / (jnp.sqrt(ssq) + _DELTA)
    v_ref[...] = jnp.dot(xi, wv_ref[...],
                         preferred_element_type=jnp.float32) + bv_ref[...]
    gates = jax.nn.sigmoid(
        jnp.dot(xi, wg_ref[...],
                preferred_element_type=jnp.float32) + bg_ref[...])
    wgate = jax.nn.sigmoid(
        jnp.dot(xi, wwg_ref[...],
                preferred_element_type=jnp.float32) + bwg_ref[...])
    ww_ref[...] = wgate * gates


def _scan_body(qn_ref, mem_ref, lu_ref, pos_ref, tv_ref, ti_ref):
    mb = pl.program_id(1)

    @pl.when(mb == 0)
    def _():
        tv_ref[...] = jnp.full((_R, _K), _NEG, jnp.float32)
        ti_ref[...] = jnp.zeros((_R, _K), jnp.int32)

    mem = mem_ref[0]
    qn = qn_ref[0]
    ssq = jnp.sum(mem * mem, axis=1, keepdims=True)
    mn = mem / (jnp.sqrt(ssq) + _DELTA)
    sims = jax.lax.dot_general(qn, mn, (((1,), (1,)), ((), ())),
                               preferred_element_type=jnp.float32)

    col = jax.lax.broadcasted_iota(jnp.int32, (_R, _BLK), 1) + mb * _BLK
    s = sims
    blk_v, blk_i = [], []
    for _ in range(_K):
        v = jnp.max(s, axis=1, keepdims=True)
        i = jnp.min(jnp.where(s == v, col, jnp.int32(2 ** 30)),
                    axis=1, keepdims=True)
        blk_v.append(v)
        blk_i.append(i)
        s = jnp.where(col == i, _NEG, s)

    cv = jnp.concatenate([tv_ref[...]] + blk_v, axis=1)
    ci = jnp.concatenate([ti_ref[...]] + blk_i, axis=1)
    col8 = jax.lax.broadcasted_iota(jnp.int32, (_R, 2 * _K), 1)
    nv, ni = [], []
    for _ in range(_K):
        v = jnp.max(cv, axis=1, keepdims=True)
        p = jnp.min(jnp.where(cv == v, col8, jnp.int32(2 * _K)),
                    axis=1, keepdims=True)
        sel = col8 == p
        i = jnp.sum(jnp.where(sel, ci, 0), axis=1, keepdims=True)
        nv.append(v)
        ni.append(i)
        cv = jnp.where(sel, _NEG, cv)
    tv_ref[...] = jnp.concatenate(nv, axis=1)
    ti_ref[...] = jnp.concatenate(ni, axis=1)

    @pl.when(mb == _NMB - 1)
    def _():
        for r in range(_R):
            pos_ref[0, 0, r * _K:(r + 1) * _K] = ti_ref[r, :]
        pos_ref[0, 0, _R * _K:_R * _K + 1] = lu_ref[0, 0, :]


def _gather_fin_body(pos_ref, wall_ref, mem_ref, qn_ref, wv_ref, out_ref,
                     vis_ref, sem):
    for b in range(_B):
        for c in range(_C):
            idx = pos_ref[b, c]
            pltpu.make_async_copy(mem_ref.at[b, idx],
                                  vis_ref.at[b * _C + c], sem).start()
    for b in range(_B):
        for c in range(_C):
            idx = pos_ref[b, c]
            pltpu.make_async_copy(mem_ref.at[b, idx],
                                  vis_ref.at[b * _C + c], sem).wait()
    for b in range(_B):
        w = wall_ref[b * _C:(b + 1) * _C, :]
        vis = vis_ref[b * _C:(b + 1) * _C, :]
        upd = vis * (1.0 - w) + w * wv_ref[b:b + 1, :]
        ssq = jnp.sum(upd * upd, axis=1, keepdims=True)
        vn = upd / (jnp.sqrt(ssq) + _DELTA)
        rs = jax.lax.dot_general(qn_ref[b], vn, (((1,), (1,)), ((), ())),
                                 preferred_element_type=jnp.float32)
        m = jnp.max(rs, axis=1, keepdims=True)
        e = jnp.exp(rs - m)
        p = e / jnp.sum(e, axis=1, keepdims=True)
        out_ref[b] = jnp.dot(p, upd, preferred_element_type=jnp.float32)


@jax.jit
def kernel(xi, memory, least_used_mem, Wq, bq, Wv, bv, Wg, bg, Wwg, bwg):
    f32 = jnp.float32
    qn_flat, wv, ww = pl.pallas_call(
        _iface_body,
        out_shape=[
            jax.ShapeDtypeStruct((_B, _R * _W), f32),
            jax.ShapeDtypeStruct((_B, _W), f32),
            jax.ShapeDtypeStruct((_B, _C), f32),
        ],
    )(xi, Wq, bq.reshape(1, -1), Wv, bv.reshape(1, -1), Wg, bg.reshape(1, -1),
      Wwg, bwg.reshape(1, -1))

    qn3 = qn_flat.reshape(_B, _R, _W)
    lu3 = least_used_mem.reshape(_B, 1, 1)

    pos3 = pl.pallas_call(
        _scan_body,
        grid=(_B, _NMB),
        in_specs=[
            pl.BlockSpec((1, _R, _W), lambda b, mb: (b, 0, 0)),
            pl.BlockSpec((1, _BLK, _W), lambda b, mb: (b, mb, 0)),
            pl.BlockSpec((1, 1, 1), lambda b, mb: (b, 0, 0)),
        ],
        out_specs=pl.BlockSpec((1, 1, _C), lambda b, mb: (b, 0, 0)),
        out_shape=jax.ShapeDtypeStruct((_B, 1, _C), jnp.int32),
        scratch_shapes=[
            pltpu.VMEM((_R, _K), f32),
            pltpu.VMEM((_R, _K), jnp.int32),
        ],
    )(qn3, memory, lu3)
    positions = pos3.reshape(_B, _C)

    read_vectors = pl.pallas_call(
        _gather_fin_body,
        in_specs=[
            pl.BlockSpec(memory_space=pltpu.SMEM),
            pl.BlockSpec(memory_space=pltpu.VMEM),
            pl.BlockSpec(memory_space=pl.ANY),
            pl.BlockSpec(memory_space=pltpu.VMEM),
            pl.BlockSpec(memory_space=pltpu.VMEM),
        ],
        out_specs=pl.BlockSpec(memory_space=pltpu.VMEM),
        out_shape=jax.ShapeDtypeStruct((_B, _R, _W), f32),
        scratch_shapes=[
            pltpu.VMEM((_B * _C, _W), f32),
            pltpu.SemaphoreType.DMA,
        ],
    )(positions, ww.reshape(_B * _C, 1), memory, qn3, wv)

    return read_vectors
